# baseline (speedup 1.0000x reference)
import functools

import jax
import jax.numpy as jnp
from jax import lax
from jax.experimental import pallas as pl
from jax.experimental.pallas import tpu as pltpu
from jax.experimental.pallas import tpu_sc as plsc

T = 2048
D = 1024
F = 2048
E = 8
B = 256
NB = 24
P = NB * B

NC = 2
NS = 16
NW = NC * NS
TPW = T // NW

EL = 2 * E
DW = D // 2


def _gate_body(x_ref, wg_ref, w0_ref, w1_ref, pos0_ref, pos1_ref, meta_ref):
    x = x_ref[...]
    wg = wg_ref[...]
    logits = jax.lax.dot_general(
        x.astype(jnp.bfloat16), wg.astype(jnp.bfloat16),
        (((1,), (0,)), ((), ())),
        preferred_element_type=jnp.float32)
    lane = jax.lax.broadcasted_iota(jnp.int32, (T, E), 1)
    m = jnp.max(logits, axis=1, keepdims=True)
    el = jnp.exp(logits - m)
    probs = el / jnp.sum(el, axis=1, keepdims=True)

    m1 = jnp.max(probs, axis=1, keepdims=True)
    a1 = jnp.min(jnp.where(probs == m1, lane, E), axis=1, keepdims=True)
    sel1 = lane == a1
    p2 = jnp.where(sel1, -1.0, probs)
    m2 = jnp.max(p2, axis=1, keepdims=True)
    a2 = jnp.min(jnp.where(p2 == m2, lane, E), axis=1, keepdims=True)

    lane2 = jax.lax.broadcasted_iota(jnp.int32, (T, EL), 1)
    sel1w = lane2 == a1
    sel2w = lane2 == (a2 + E)
    oh = (sel1w | sel2w).astype(jnp.bfloat16)
    r_i = jax.lax.broadcasted_iota(jnp.int32, (T, T), 0)
    c_i = jax.lax.broadcasted_iota(jnp.int32, (T, T), 1)
    tril = (c_i <= r_i).astype(jnp.bfloat16)
    c = jax.lax.dot_general(
        tril, oh, (((1,), (0,)), ((), ())),
        preferred_element_type=jnp.float32)

    clast = c[T - 1:T, :]
    li = jax.lax.broadcasted_iota(jnp.int32, (EL, EL), 0)
    lj = jax.lax.broadcasted_iota(jnp.int32, (EL, EL), 1)
    shift_dn = (li == lj + E).astype(jnp.float32)
    shift_up = (lj == li + E).astype(jnp.float32)
    strict_lt = ((li < lj) & (li < E)).astype(jnp.float32)

    def lmm(v, mat):
        return jax.lax.dot_general(v, mat, (((1,), (0,)), ((), ())),
                                   precision=jax.lax.Precision.HIGHEST,
                                   preferred_element_type=jnp.float32)

    lane1 = jax.lax.broadcasted_iota(jnp.int32, (1, EL), 1)
    n1 = jnp.where(lane1 < E, clast, 0.0)
    n2a = lmm(clast, shift_dn)
    cnt = n1 + n2a
    bpad = jnp.float32(B)
    padded = jnp.ceil(cnt / bpad) * bpad
    base = lmm(padded, strict_lt)
    n1s = lmm(n1, shift_up)
    bases = lmm(base, shift_up)

    pos0 = jnp.sum(jnp.where(sel1w, c - 1.0 + base, 0.0), axis=1,
                   keepdims=True)
    pos1 = jnp.sum(jnp.where(sel2w, c - 1.0 + n1s + bases, 0.0), axis=1,
                   keepdims=True)

    nrep = padded / bpad
    cum_b = lmm(nrep, ((li <= lj) & (li < E)).astype(jnp.float32))
    ml = jax.lax.broadcasted_iota(jnp.int32, (1, 64), 1)
    nb_s = jnp.sum(jnp.where(lane1 == E - 1, cum_b, 0.0), axis=1,
                   keepdims=True)
    cc = jnp.broadcast_to(cum_b, (64, EL))
    ji = jax.lax.broadcasted_iota(jnp.int32, (64, EL), 0)
    ind = ((cc == ji.astype(jnp.float32)) &
           (jax.lax.broadcasted_iota(jnp.int32, (64, EL), 1) < E)
           ).astype(jnp.float32)
    v = jnp.sum(ind, axis=1, keepdims=True)
    li32 = jax.lax.broadcasted_iota(jnp.int32, (64, 64), 0)
    lj32 = jax.lax.broadcasted_iota(jnp.int32, (64, 64), 1)
    bec = jax.lax.dot_general(
        v, (li32 <= lj32).astype(jnp.float32), (((0,), (0,)), ((), ())),
        precision=jax.lax.Precision.HIGHEST,
        preferred_element_type=jnp.float32)
    bec = jnp.minimum(bec, float(E - 1))
    active = ml.astype(jnp.float32) < nb_s
    be_last = jnp.max(jnp.where(active, bec, 0.0), axis=1, keepdims=True)
    bef = jnp.where(active, bec, be_last)
    meta = jnp.where(ml == NB, nb_s, bef)

    w0_ref[...] = jnp.broadcast_to(m1, (T, 16))
    w1_ref[...] = jnp.broadcast_to(m2, (T, 16))
    pos0_ref[...] = pos0.astype(jnp.int32)
    pos1_ref[...] = pos1.astype(jnp.int32)
    meta_ref[...] = meta.astype(jnp.int32)


def _gate_call(x, wg, interpret=False):
    return pl.pallas_call(
        _gate_body,
        out_shape=(
            jax.ShapeDtypeStruct((T, 16), jnp.float32),
            jax.ShapeDtypeStruct((T, 16), jnp.float32),
            jax.ShapeDtypeStruct((T, 1), jnp.int32),
            jax.ShapeDtypeStruct((T, 1), jnp.int32),
            jax.ShapeDtypeStruct((1, 64), jnp.int32),
        ),
        interpret=interpret,
    )(x, wg)


def _dispatch_call(x, pos0, pos1):
    mesh = plsc.VectorSubcoreMesh(core_axis_name="c", subcore_axis_name="s")

    @functools.partial(
        pl.kernel,
        out_type=jax.ShapeDtypeStruct((P, D), jnp.float32),
        mesh=mesh,
        scratch_types=[
            pltpu.VMEM((TPW,), jnp.int32),
            pltpu.VMEM((TPW,), jnp.int32),
            pltpu.VMEM((TPW, D), jnp.float32),
            pltpu.SemaphoreType.DMA,
        ],
    )
    def dispatch(x_hbm, pos0_hbm, pos1_hbm, xs_hbm, idx0_v, idx1_v, rows_v,
                 sem):
        wid = lax.axis_index("s") * NC + lax.axis_index("c")
        t0 = wid * TPW
        pltpu.sync_copy(x_hbm.at[pl.ds(t0, TPW)], rows_v)
        pltpu.sync_copy(pos0_hbm.at[pl.ds(t0, TPW)], idx0_v)
        pltpu.sync_copy(pos1_hbm.at[pl.ds(t0, TPW)], idx1_v)
        pltpu.async_copy(rows_v, xs_hbm.at[idx0_v], sem).wait()
        pltpu.async_copy(rows_v, xs_hbm.at[idx1_v], sem).wait()

    return dispatch(x, pos0, pos1)


def _ffn_body(be_ref, nb_ref, xs_ref, w1_ref, w2_ref, ys_ref):
    i = pl.program_id(0)

    @pl.when(i < nb_ref[0])
    def _():
        xb = xs_ref[...].astype(jnp.bfloat16)
        h = jax.lax.dot_general(
            xb, w1_ref[0].astype(jnp.bfloat16), (((1,), (0,)), ((), ())),
            preferred_element_type=jnp.float32)
        h = jnp.maximum(h, 0.0).astype(jnp.bfloat16)
        y = jax.lax.dot_general(
            h, w2_ref[0].astype(jnp.bfloat16), (((1,), (0,)), ((), ())),
            preferred_element_type=jnp.float32)
        ys_ref[...] = y


def _ffn_call(be, nb, xs, w1, w2, interpret=False):
    grid_spec = pltpu.PrefetchScalarGridSpec(
        num_scalar_prefetch=2,
        grid=(NB,),
        in_specs=[
            pl.BlockSpec((B, D),
                         lambda i, be, nb: (jnp.minimum(i, nb[0] - 1), 0)),
            pl.BlockSpec((1, D, F), lambda i, be, nb: (be[i], 0, 0)),
            pl.BlockSpec((1, F, D), lambda i, be, nb: (be[i], 0, 0)),
        ],
        out_specs=pl.BlockSpec(
            (B, D), lambda i, be, nb: (jnp.minimum(i, nb[0] - 1), 0)),
    )
    return pl.pallas_call(
        _ffn_body,
        grid_spec=grid_spec,
        out_shape=jax.ShapeDtypeStruct((P, D), jnp.float32),
        interpret=interpret,
    )(be, nb, xs, w1, w2)


HT = 32


def _combine_call(ys, pos0, pos1, w0, w1):
    mesh = plsc.VectorSubcoreMesh(core_axis_name="c", subcore_axis_name="s")

    @functools.partial(
        pl.kernel,
        out_type=jax.ShapeDtypeStruct((T, D), jnp.float32),
        mesh=mesh,
        scratch_types=[
            pltpu.VMEM((HT,), jnp.int32),
            pltpu.VMEM((HT, 16), jnp.float32),
            pltpu.VMEM((HT, 16), jnp.float32),
            pltpu.VMEM((HT, D), jnp.float32),
            pltpu.VMEM((HT, D), jnp.float32),
            pltpu.VMEM((HT, D), jnp.float32),
            pltpu.SemaphoreType.DMA,
        ],
    )
    def combine(ys_hbm, pos0_hbm, pos1_hbm, w0_hbm, w1_hbm, out_hbm,
                idx_v, w0_v, w1_v, a_v, b_v, o_v, sem):
        wid = lax.axis_index("s") * NC + lax.axis_index("c")
        t0 = wid * TPW
        for h in range(TPW // HT):
            tb = t0 + h * HT
            pltpu.sync_copy(pos0_hbm.at[pl.ds(tb, HT)], idx_v)
            pltpu.async_copy(ys_hbm.at[idx_v], a_v, sem).wait()
            pltpu.sync_copy(pos1_hbm.at[pl.ds(tb, HT)], idx_v)
            pltpu.async_copy(ys_hbm.at[idx_v], b_v, sem).wait()
            pltpu.sync_copy(w0_hbm.at[pl.ds(tb, HT)], w0_v)
            pltpu.sync_copy(w1_hbm.at[pl.ds(tb, HT)], w1_v)

            def outer(r, carry):
                wv0 = w0_v[r, :]
                wv1 = w1_v[r, :]

                def inner(q, carry2):
                    o_v[r, pl.ds(q * 16, 16)] = (
                        a_v[r, pl.ds(q * 16, 16)] * wv0 +
                        b_v[r, pl.ds(q * 16, 16)] * wv1)
                    return carry2

                lax.fori_loop(0, D // 16, inner, 0, unroll=8)
                return carry

            lax.fori_loop(0, HT, outer, 0)
            pltpu.sync_copy(o_v, out_hbm.at[pl.ds(tb, HT)])

    return combine(ys, pos0, pos1, w0, w1)


def kernel(input, Wg, W1, W2):
    x = input
    w0, w1, pos0, pos1, meta = _gate_call(x, Wg)

    be = meta[0, :NB]
    nb = meta[0, NB:NB + 1]
    pos0f = pos0.reshape(T)
    pos1f = pos1.reshape(T)
    xs = _dispatch_call(x, pos0f, pos1f)
    ys = _ffn_call(be, nb, xs, W1, W2)
    return _combine_call(ys, pos0f, pos1f, w0, w1)


# speedup vs baseline: 1.1183x; 1.1183x over previous
import functools

import jax
import jax.numpy as jnp
from jax import lax
from jax.experimental import pallas as pl
from jax.experimental.pallas import tpu as pltpu
from jax.experimental.pallas import tpu_sc as plsc

T = 2048
D = 1024
F = 2048
E = 8
B = 256
NB = 24
P = NB * B

NC = 2
NS = 16
NW = NC * NS
TPW = T // NW

EL = 2 * E
DW = D // 2


def _gate_body(x_ref, wg_ref, w0_ref, w1_ref, pos0_ref, pos1_ref, meta_ref):
    x = x_ref[...]
    wg = wg_ref[...]
    logits = jax.lax.dot_general(
        x.astype(jnp.bfloat16), wg.astype(jnp.bfloat16),
        (((1,), (0,)), ((), ())),
        preferred_element_type=jnp.float32)
    lane = jax.lax.broadcasted_iota(jnp.int32, (T, E), 1)
    m = jnp.max(logits, axis=1, keepdims=True)
    el = jnp.exp(logits - m)
    probs = el / jnp.sum(el, axis=1, keepdims=True)

    m1 = jnp.max(probs, axis=1, keepdims=True)
    a1 = jnp.min(jnp.where(probs == m1, lane, E), axis=1, keepdims=True)
    sel1 = lane == a1
    p2 = jnp.where(sel1, -1.0, probs)
    m2 = jnp.max(p2, axis=1, keepdims=True)
    a2 = jnp.min(jnp.where(p2 == m2, lane, E), axis=1, keepdims=True)

    lane2 = jax.lax.broadcasted_iota(jnp.int32, (T, EL), 1)
    sel1w = lane2 == a1
    sel2w = lane2 == (a2 + E)
    oh = (sel1w | sel2w).astype(jnp.bfloat16)
    r_i = jax.lax.broadcasted_iota(jnp.int32, (T, T), 0)
    c_i = jax.lax.broadcasted_iota(jnp.int32, (T, T), 1)
    tril = (c_i <= r_i).astype(jnp.bfloat16)
    c = jax.lax.dot_general(
        tril, oh, (((1,), (0,)), ((), ())),
        preferred_element_type=jnp.float32)

    clast = c[T - 1:T, :]
    li = jax.lax.broadcasted_iota(jnp.int32, (EL, EL), 0)
    lj = jax.lax.broadcasted_iota(jnp.int32, (EL, EL), 1)
    shift_dn = (li == lj + E).astype(jnp.float32)
    shift_up = (lj == li + E).astype(jnp.float32)
    strict_lt = ((li < lj) & (li < E)).astype(jnp.float32)

    def lmm(v, mat):
        return jax.lax.dot_general(v, mat, (((1,), (0,)), ((), ())),
                                   precision=jax.lax.Precision.HIGHEST,
                                   preferred_element_type=jnp.float32)

    lane1 = jax.lax.broadcasted_iota(jnp.int32, (1, EL), 1)
    n1 = jnp.where(lane1 < E, clast, 0.0)
    n2a = lmm(clast, shift_dn)
    cnt = n1 + n2a
    bpad = jnp.float32(B)
    padded = jnp.ceil(cnt / bpad) * bpad
    base = lmm(padded, strict_lt)
    n1s = lmm(n1, shift_up)
    bases = lmm(base, shift_up)

    pos0 = jnp.sum(jnp.where(sel1w, c - 1.0 + base, 0.0), axis=1,
                   keepdims=True)
    pos1 = jnp.sum(jnp.where(sel2w, c - 1.0 + n1s + bases, 0.0), axis=1,
                   keepdims=True)

    nrep = padded / bpad
    cum_b = lmm(nrep, ((li <= lj) & (li < E)).astype(jnp.float32))
    ml = jax.lax.broadcasted_iota(jnp.int32, (1, 64), 1)
    nb_s = jnp.sum(jnp.where(lane1 == E - 1, cum_b, 0.0), axis=1,
                   keepdims=True)
    cc = jnp.broadcast_to(cum_b, (64, EL))
    ji = jax.lax.broadcasted_iota(jnp.int32, (64, EL), 0)
    ind = ((cc == ji.astype(jnp.float32)) &
           (jax.lax.broadcasted_iota(jnp.int32, (64, EL), 1) < E)
           ).astype(jnp.float32)
    v = jnp.sum(ind, axis=1, keepdims=True)
    li32 = jax.lax.broadcasted_iota(jnp.int32, (64, 64), 0)
    lj32 = jax.lax.broadcasted_iota(jnp.int32, (64, 64), 1)
    bec = jax.lax.dot_general(
        v, (li32 <= lj32).astype(jnp.float32), (((0,), (0,)), ((), ())),
        precision=jax.lax.Precision.HIGHEST,
        preferred_element_type=jnp.float32)
    bec = jnp.minimum(bec, float(E - 1))
    active = ml.astype(jnp.float32) < nb_s
    be_last = jnp.max(jnp.where(active, bec, 0.0), axis=1, keepdims=True)
    bef = jnp.where(active, bec, be_last)
    meta = jnp.where(ml == NB, nb_s, bef)

    w0_ref[...] = m1
    w1_ref[...] = m2
    pos0_ref[...] = pos0.astype(jnp.int32)
    pos1_ref[...] = pos1.astype(jnp.int32)
    meta_ref[...] = meta.astype(jnp.int32)


def _gate_call(x, wg, interpret=False):
    return pl.pallas_call(
        _gate_body,
        out_shape=(
            jax.ShapeDtypeStruct((T, 1), jnp.float32),
            jax.ShapeDtypeStruct((T, 1), jnp.float32),
            jax.ShapeDtypeStruct((T, 1), jnp.int32),
            jax.ShapeDtypeStruct((T, 1), jnp.int32),
            jax.ShapeDtypeStruct((1, 64), jnp.int32),
        ),
        interpret=interpret,
    )(x, wg)


def _dispatch_call(x, pos0, pos1):
    mesh = plsc.VectorSubcoreMesh(core_axis_name="c", subcore_axis_name="s")

    @functools.partial(
        pl.kernel,
        out_type=jax.ShapeDtypeStruct((P, D), jnp.float32),
        mesh=mesh,
        scratch_types=[
            pltpu.VMEM((TPW,), jnp.int32),
            pltpu.VMEM((TPW,), jnp.int32),
            pltpu.VMEM((TPW, D), jnp.float32),
            pltpu.SemaphoreType.DMA,
        ],
    )
    def dispatch(x_hbm, pos0_hbm, pos1_hbm, xs_hbm, idx0_v, idx1_v, rows_v,
                 sem):
        wid = lax.axis_index("s") * NC + lax.axis_index("c")
        t0 = wid * TPW
        pltpu.sync_copy(x_hbm.at[pl.ds(t0, TPW)], rows_v)
        pltpu.sync_copy(pos0_hbm.at[pl.ds(t0, TPW)], idx0_v)
        pltpu.sync_copy(pos1_hbm.at[pl.ds(t0, TPW)], idx1_v)
        pltpu.async_copy(rows_v, xs_hbm.at[idx0_v], sem).wait()
        pltpu.async_copy(rows_v, xs_hbm.at[idx1_v], sem).wait()

    return dispatch(x, pos0, pos1)


def _ffn_body(be_ref, nb_ref, xs_ref, w1_ref, w2_ref, ys_ref):
    i = pl.program_id(0)

    @pl.when(i < nb_ref[0])
    def _():
        xb = xs_ref[...].astype(jnp.bfloat16)
        h = jax.lax.dot_general(
            xb, w1_ref[0].astype(jnp.bfloat16), (((1,), (0,)), ((), ())),
            preferred_element_type=jnp.float32)
        h = jnp.maximum(h, 0.0).astype(jnp.bfloat16)
        y = jax.lax.dot_general(
            h, w2_ref[0].astype(jnp.bfloat16), (((1,), (0,)), ((), ())),
            preferred_element_type=jnp.float32)
        ys_ref[...] = y


def _ffn_call(be, nb, xs, w1, w2, interpret=False):
    grid_spec = pltpu.PrefetchScalarGridSpec(
        num_scalar_prefetch=2,
        grid=(NB,),
        in_specs=[
            pl.BlockSpec((B, D),
                         lambda i, be, nb: (jnp.minimum(i, nb[0] - 1), 0)),
            pl.BlockSpec((1, D, F), lambda i, be, nb: (be[i], 0, 0)),
            pl.BlockSpec((1, F, D), lambda i, be, nb: (be[i], 0, 0)),
        ],
        out_specs=pl.BlockSpec(
            (B, D), lambda i, be, nb: (jnp.minimum(i, nb[0] - 1), 0)),
    )
    return pl.pallas_call(
        _ffn_body,
        grid_spec=grid_spec,
        out_shape=jax.ShapeDtypeStruct((P, D), jnp.float32),
        interpret=interpret,
    )(be, nb, xs, w1, w2)


def _combine_call(ys, pos0, pos1):
    mesh = plsc.VectorSubcoreMesh(core_axis_name="c", subcore_axis_name="s")

    @functools.partial(
        pl.kernel,
        out_type=(
            jax.ShapeDtypeStruct((T, D), jnp.float32),
            jax.ShapeDtypeStruct((T, D), jnp.float32),
        ),
        mesh=mesh,
        scratch_types=[
            pltpu.VMEM((TPW,), jnp.int32),
            pltpu.VMEM((TPW, D), jnp.float32),
            pltpu.SemaphoreType.DMA,
        ],
    )
    def combine(ys_hbm, pos0_hbm, pos1_hbm, y0_hbm, y1_hbm, idx_v, rows_v,
                sem):
        wid = lax.axis_index("s") * NC + lax.axis_index("c")
        t0 = wid * TPW
        pltpu.sync_copy(pos0_hbm.at[pl.ds(t0, TPW)], idx_v)
        pltpu.async_copy(ys_hbm.at[idx_v], rows_v, sem).wait()
        pltpu.sync_copy(rows_v, y0_hbm.at[pl.ds(t0, TPW)])
        pltpu.sync_copy(pos1_hbm.at[pl.ds(t0, TPW)], idx_v)
        pltpu.async_copy(ys_hbm.at[idx_v], rows_v, sem).wait()
        pltpu.sync_copy(rows_v, y1_hbm.at[pl.ds(t0, TPW)])

    return combine(ys, pos0, pos1)


def _wsum_body(w0_ref, w1_ref, y0_ref, y1_ref, out_ref):
    out_ref[...] = (w0_ref[...] * y0_ref[...] + w1_ref[...] * y1_ref[...])


def _wsum_call(w0, w1, y0, y1, interpret=False):
    nblk = 4
    rb = T // nblk
    return pl.pallas_call(
        _wsum_body,
        grid=(nblk,),
        in_specs=[
            pl.BlockSpec((rb, 1), lambda i: (i, 0)),
            pl.BlockSpec((rb, 1), lambda i: (i, 0)),
            pl.BlockSpec((rb, D), lambda i: (i, 0)),
            pl.BlockSpec((rb, D), lambda i: (i, 0)),
        ],
        out_specs=pl.BlockSpec((rb, D), lambda i: (i, 0)),
        out_shape=jax.ShapeDtypeStruct((T, D), jnp.float32),
        interpret=interpret,
    )(w0, w1, y0, y1)


def kernel(input, Wg, W1, W2):
    x = input
    w0, w1, pos0, pos1, meta = _gate_call(x, Wg)

    be = meta[0, :NB]
    nb = meta[0, NB:NB + 1]
    pos0f = pos0.reshape(T)
    pos1f = pos1.reshape(T)
    xs = _dispatch_call(x, pos0f, pos1f)
    ys = _ffn_call(be, nb, xs, W1, W2)
    y0, y1 = _combine_call(ys, pos0f, pos1f)
    return _wsum_call(w0, w1, y0, y1)


# speedup vs baseline: 1.1192x; 1.0008x over previous
import functools

import jax
import jax.numpy as jnp
from jax import lax
from jax.experimental import pallas as pl
from jax.experimental.pallas import tpu as pltpu
from jax.experimental.pallas import tpu_sc as plsc

T = 2048
D = 1024
F = 2048
E = 8
B = 256
NB = 24
P = NB * B

NC = 2
NS = 16
NW = NC * NS
TPW = T // NW

EL = 2 * E
DW = D // 2


def _gate_body(x_ref, wg_ref, w0_ref, w1_ref, pos0_ref, pos1_ref, meta_ref):
    x = x_ref[...]
    wg = wg_ref[...]
    logits = jax.lax.dot_general(
        x.astype(jnp.bfloat16), wg.astype(jnp.bfloat16),
        (((1,), (0,)), ((), ())),
        preferred_element_type=jnp.float32)
    lane = jax.lax.broadcasted_iota(jnp.int32, (T, E), 1)
    m = jnp.max(logits, axis=1, keepdims=True)
    el = jnp.exp(logits - m)
    probs = el / jnp.sum(el, axis=1, keepdims=True)

    m1 = jnp.max(probs, axis=1, keepdims=True)
    a1 = jnp.min(jnp.where(probs == m1, lane, E), axis=1, keepdims=True)
    sel1 = lane == a1
    p2 = jnp.where(sel1, -1.0, probs)
    m2 = jnp.max(p2, axis=1, keepdims=True)
    a2 = jnp.min(jnp.where(p2 == m2, lane, E), axis=1, keepdims=True)

    lane2 = jax.lax.broadcasted_iota(jnp.int32, (T, EL), 1)
    sel1w = lane2 == a1
    sel2w = lane2 == (a2 + E)
    oh = (sel1w | sel2w).astype(jnp.bfloat16)
    r_i = jax.lax.broadcasted_iota(jnp.int32, (T, T), 0)
    c_i = jax.lax.broadcasted_iota(jnp.int32, (T, T), 1)
    tril = (c_i <= r_i).astype(jnp.bfloat16)
    c = jax.lax.dot_general(
        tril, oh, (((1,), (0,)), ((), ())),
        preferred_element_type=jnp.float32)

    clast = c[T - 1:T, :]
    li = jax.lax.broadcasted_iota(jnp.int32, (EL, EL), 0)
    lj = jax.lax.broadcasted_iota(jnp.int32, (EL, EL), 1)
    shift_dn = (li == lj + E).astype(jnp.float32)
    shift_up = (lj == li + E).astype(jnp.float32)
    strict_lt = ((li < lj) & (li < E)).astype(jnp.float32)

    def lmm(v, mat):
        return jax.lax.dot_general(v, mat, (((1,), (0,)), ((), ())),
                                   precision=jax.lax.Precision.HIGHEST,
                                   preferred_element_type=jnp.float32)

    lane1 = jax.lax.broadcasted_iota(jnp.int32, (1, EL), 1)
    n1 = jnp.where(lane1 < E, clast, 0.0)
    n2a = lmm(clast, shift_dn)
    cnt = n1 + n2a
    bpad = jnp.float32(B)
    padded = jnp.ceil(cnt / bpad) * bpad
    base = lmm(padded, strict_lt)
    n1s = lmm(n1, shift_up)
    bases = lmm(base, shift_up)

    pos0 = jnp.sum(jnp.where(sel1w, c - 1.0 + base, 0.0), axis=1,
                   keepdims=True)
    pos1 = jnp.sum(jnp.where(sel2w, c - 1.0 + n1s + bases, 0.0), axis=1,
                   keepdims=True)

    nrep = padded / bpad
    cum_b = lmm(nrep, ((li <= lj) & (li < E)).astype(jnp.float32))
    ml = jax.lax.broadcasted_iota(jnp.int32, (1, 64), 1)
    nb_s = jnp.sum(jnp.where(lane1 == E - 1, cum_b, 0.0), axis=1,
                   keepdims=True)
    cc = jnp.broadcast_to(cum_b, (64, EL))
    ji = jax.lax.broadcasted_iota(jnp.int32, (64, EL), 0)
    ind = ((cc == ji.astype(jnp.float32)) &
           (jax.lax.broadcasted_iota(jnp.int32, (64, EL), 1) < E)
           ).astype(jnp.float32)
    v = jnp.sum(ind, axis=1, keepdims=True)
    li32 = jax.lax.broadcasted_iota(jnp.int32, (64, 64), 0)
    lj32 = jax.lax.broadcasted_iota(jnp.int32, (64, 64), 1)
    bec = jax.lax.dot_general(
        v, (li32 <= lj32).astype(jnp.float32), (((0,), (0,)), ((), ())),
        precision=jax.lax.Precision.HIGHEST,
        preferred_element_type=jnp.float32)
    bec = jnp.minimum(bec, float(E - 1))
    active = ml.astype(jnp.float32) < nb_s
    be_last = jnp.max(jnp.where(active, bec, 0.0), axis=1, keepdims=True)
    bef = jnp.where(active, bec, be_last)
    meta = jnp.where(ml == NB, nb_s, bef)

    w0_ref[...] = m1
    w1_ref[...] = m2
    pos0_ref[...] = pos0.astype(jnp.int32)
    pos1_ref[...] = pos1.astype(jnp.int32)
    meta_ref[...] = meta.astype(jnp.int32)


def _gate_call(x, wg):
    return pl.pallas_call(
        _gate_body,
        out_shape=(
            jax.ShapeDtypeStruct((T, 1), jnp.float32),
            jax.ShapeDtypeStruct((T, 1), jnp.float32),
            jax.ShapeDtypeStruct((T, 1), jnp.int32),
            jax.ShapeDtypeStruct((T, 1), jnp.int32),
            jax.ShapeDtypeStruct((1, 64), jnp.int32),
        ),
    )(x, wg)


def _dispatch_call(x, pos0, pos1):
    mesh = plsc.VectorSubcoreMesh(core_axis_name="c", subcore_axis_name="s")

    @functools.partial(
        pl.kernel,
        out_type=jax.ShapeDtypeStruct((P, D), jnp.float32),
        mesh=mesh,
        scratch_types=[
            pltpu.VMEM((TPW,), jnp.int32),
            pltpu.VMEM((TPW,), jnp.int32),
            pltpu.VMEM((TPW, D), jnp.float32),
            pltpu.SemaphoreType.DMA,
        ],
    )
    def dispatch(x_hbm, pos0_hbm, pos1_hbm, xs_hbm, idx0_v, idx1_v, rows_v,
                 sem):
        wid = lax.axis_index("s") * NC + lax.axis_index("c")
        t0 = wid * TPW
        pltpu.sync_copy(x_hbm.at[pl.ds(t0, TPW)], rows_v)
        pltpu.sync_copy(pos0_hbm.at[pl.ds(t0, TPW)], idx0_v)
        pltpu.sync_copy(pos1_hbm.at[pl.ds(t0, TPW)], idx1_v)
        pltpu.async_copy(rows_v, xs_hbm.at[idx0_v], sem).wait()
        pltpu.async_copy(rows_v, xs_hbm.at[idx1_v], sem).wait()

    return dispatch(x, pos0, pos1)


def _ffn_body(be_ref, nb_ref, xs_ref, w1_ref, w2_ref, ys_ref):
    i = pl.program_id(0)

    @pl.when(i < nb_ref[0])
    def _():
        xb = xs_ref[...].astype(jnp.bfloat16)
        h = jax.lax.dot_general(
            xb, w1_ref[0].astype(jnp.bfloat16), (((1,), (0,)), ((), ())),
            preferred_element_type=jnp.float32)
        h = jnp.maximum(h, 0.0).astype(jnp.bfloat16)
        y = jax.lax.dot_general(
            h, w2_ref[0].astype(jnp.bfloat16), (((1,), (0,)), ((), ())),
            preferred_element_type=jnp.float32)
        ys_ref[...] = y


def _ffn_call(be, nb, xs, w1, w2):
    grid_spec = pltpu.PrefetchScalarGridSpec(
        num_scalar_prefetch=2,
        grid=(NB,),
        in_specs=[
            pl.BlockSpec((B, D),
                         lambda i, be, nb: (jnp.minimum(i, nb[0] - 1), 0)),
            pl.BlockSpec((1, D, F), lambda i, be, nb: (be[i], 0, 0)),
            pl.BlockSpec((1, F, D), lambda i, be, nb: (be[i], 0, 0)),
        ],
        out_specs=pl.BlockSpec(
            (B, D), lambda i, be, nb: (jnp.minimum(i, nb[0] - 1), 0)),
    )
    return pl.pallas_call(
        _ffn_body,
        grid_spec=grid_spec,
        out_shape=jax.ShapeDtypeStruct((P, D), jnp.float32),
    )(be, nb, xs, w1, w2)


def _combine_call(ys, pos0, pos1):
    mesh = plsc.VectorSubcoreMesh(core_axis_name="c", subcore_axis_name="s")

    @functools.partial(
        pl.kernel,
        out_type=(
            jax.ShapeDtypeStruct((T, D), jnp.float32),
            jax.ShapeDtypeStruct((T, D), jnp.float32),
        ),
        mesh=mesh,
        scratch_types=[
            pltpu.VMEM((TPW,), jnp.int32),
            pltpu.VMEM((TPW, D), jnp.float32),
            pltpu.SemaphoreType.DMA,
        ],
    )
    def combine(ys_hbm, pos0_hbm, pos1_hbm, y0_hbm, y1_hbm, idx_v, rows_v,
                sem):
        wid = lax.axis_index("s") * NC + lax.axis_index("c")
        t0 = wid * TPW
        pltpu.sync_copy(pos0_hbm.at[pl.ds(t0, TPW)], idx_v)
        pltpu.async_copy(ys_hbm.at[idx_v], rows_v, sem).wait()
        pltpu.sync_copy(rows_v, y0_hbm.at[pl.ds(t0, TPW)])
        pltpu.sync_copy(pos1_hbm.at[pl.ds(t0, TPW)], idx_v)
        pltpu.async_copy(ys_hbm.at[idx_v], rows_v, sem).wait()
        pltpu.sync_copy(rows_v, y1_hbm.at[pl.ds(t0, TPW)])

    return combine(ys, pos0, pos1)


def _wsum_body(w0_ref, w1_ref, y0_ref, y1_ref, out_ref):
    out_ref[...] = (w0_ref[...] * y0_ref[...] + w1_ref[...] * y1_ref[...])


def _wsum_call(w0, w1, y0, y1):
    nblk = 4
    rb = T // nblk
    return pl.pallas_call(
        _wsum_body,
        grid=(nblk,),
        in_specs=[
            pl.BlockSpec((rb, 1), lambda i: (i, 0)),
            pl.BlockSpec((rb, 1), lambda i: (i, 0)),
            pl.BlockSpec((rb, D), lambda i: (i, 0)),
            pl.BlockSpec((rb, D), lambda i: (i, 0)),
        ],
        out_specs=pl.BlockSpec((rb, D), lambda i: (i, 0)),
        out_shape=jax.ShapeDtypeStruct((T, D), jnp.float32),
    )(w0, w1, y0, y1)


def kernel(input, Wg, W1, W2):
    x = input
    w0, w1, pos0, pos1, meta = _gate_call(x, Wg)

    be = meta[0, :NB]
    nb = meta[0, NB:NB + 1]
    pos0f = pos0.reshape(T)
    pos1f = pos1.reshape(T)
    xs = _dispatch_call(x, pos0f, pos1f)
    ys = _ffn_call(be, nb, xs, W1, W2)
    y0, y1 = _combine_call(ys, pos0f, pos1f)
    return _wsum_call(w0, w1, y0, y1)
